# Initial kernel scaffold; baseline (speedup 1.0000x reference)
import jax
import jax.numpy as jnp
from jax.experimental import pallas as pl


def kernel(ab_x, ab_edge_index, ab_batch, ba_x, ba_edge_index, ba_edge_attr, ba_batch, W1, b1, W2, b2, gamma, beta):
    raise NotImplementedError("write your pallas kernel here")


# speedup vs baseline: 1.0570x; 1.0570x over previous
import functools

import jax
import jax.numpy as jnp
from jax import lax
from jax.experimental import pallas as pl
from jax.experimental.pallas import tpu as pltpu

EMB = 128
HID = 256
NGRP = 256
LAYERS = 3


def _counts_body(batch_ref, row_ref, col_ref):
    i = pl.program_id(0)

    @pl.when(i == 0)
    def _():
        row_ref[...] = jnp.zeros_like(row_ref)
        col_ref[...] = jnp.zeros_like(col_ref)

    b = batch_ref[...]
    onehot = (b == lax.broadcasted_iota(jnp.int32, (1, NGRP), 1)).astype(jnp.float32)
    row_ref[0:1, :] += jnp.sum(onehot, axis=0, keepdims=True)
    col_ref[...] += lax.dot_general(
        onehot, jnp.ones((onehot.shape[0], 128), jnp.float32),
        (((0,), (0,)), ((), ())), preferred_element_type=jnp.float32)


def _counts(batch2d, interpret=False):
    n = batch2d.shape[0]
    B = 2048
    npad = pl.cdiv(n, B) * B
    bpad = jnp.full((npad, 1), NGRP, jnp.int32).at[:n].set(batch2d)
    return pl.pallas_call(
        _counts_body,
        grid=(npad // B,),
        in_specs=[pl.BlockSpec((B, 1), lambda i: (i, 0))],
        out_specs=[pl.BlockSpec((8, NGRP), lambda i: (0, 0)),
                   pl.BlockSpec((NGRP, 128), lambda i: (0, 0))],
        out_shape=[jax.ShapeDtypeStruct((8, NGRP), jnp.float32),
                   jax.ShapeDtypeStruct((NGRP, 128), jnp.float32)],
        interpret=interpret,
    )(bpad)


def _scale_body(batch_ref, cmat_ref, o_ref):
    b = batch_ref[...]
    onehot = (b == lax.broadcasted_iota(jnp.int32, (1, NGRP), 1)).astype(jnp.float32)
    rs = lax.rsqrt(jnp.maximum(cmat_ref[...][:, 0:1], 1.0))
    o_ref[...] = lax.dot_general(onehot, rs, (((1,), (0,)), ((), ())),
                                 preferred_element_type=jnp.float32)


def _node_scale(batch2d, cmat, interpret=False):
    n = batch2d.shape[0]
    B = 2048
    npad = pl.cdiv(n, B) * B
    bpad = jnp.full((npad, 1), NGRP, jnp.int32).at[:n].set(batch2d)
    out = pl.pallas_call(
        _scale_body,
        grid=(npad // B,),
        in_specs=[pl.BlockSpec((B, 1), lambda i: (i, 0)),
                  pl.BlockSpec((NGRP, 128), lambda i: (0, 0))],
        out_specs=pl.BlockSpec((B, 1), lambda i: (i, 0)),
        out_shape=jax.ShapeDtypeStruct((npad, 1), jnp.float32),
        interpret=interpret,
    )(bpad, cmat)
    return out[:n]


def _mlp_body(agg_ref, x_ref, scale_ref, W1_ref, b1_ref, W2_ref, b2_ref,
              g_ref, be_ref, o_ref, *, last_act):
    agg = agg_ref[...]
    u = lax.dot_general(agg, W1_ref[...], (((1,), (0,)), ((), ())),
                        preferred_element_type=jnp.float32) + b1_ref[...]
    u = jnp.maximum(u, 0.0)
    h = lax.dot_general(u, W2_ref[...], (((1,), (0,)), ((), ())),
                        preferred_element_type=jnp.float32) + b2_ref[...]
    mu = jnp.mean(h, axis=1, keepdims=True)
    var = jnp.mean((h - mu) * (h - mu), axis=1, keepdims=True)
    h = (h - mu) * lax.rsqrt(var + 1e-5) * g_ref[...] + be_ref[...]
    h = h * scale_ref[...]
    if last_act:
        h = jnp.maximum(h, 0.0)
    o_ref[...] = h + x_ref[...]


def _mlp(agg, x, scale, W1, b1, W2, b2, g, be, last_act, interpret=False):
    n = agg.shape[0]
    B = 1024
    body = functools.partial(_mlp_body, last_act=last_act)
    return pl.pallas_call(
        body,
        grid=(pl.cdiv(n, B),),
        in_specs=[pl.BlockSpec((B, EMB), lambda i: (i, 0)),
                  pl.BlockSpec((B, EMB), lambda i: (i, 0)),
                  pl.BlockSpec((B, 1), lambda i: (i, 0)),
                  pl.BlockSpec((EMB, HID), lambda i: (0, 0)),
                  pl.BlockSpec((1, HID), lambda i: (0, 0)),
                  pl.BlockSpec((HID, EMB), lambda i: (0, 0)),
                  pl.BlockSpec((1, EMB), lambda i: (0, 0)),
                  pl.BlockSpec((1, EMB), lambda i: (0, 0)),
                  pl.BlockSpec((1, EMB), lambda i: (0, 0))],
        out_specs=pl.BlockSpec((B, EMB), lambda i: (i, 0)),
        out_shape=jax.ShapeDtypeStruct((n, EMB), jnp.float32),
        interpret=interpret,
    )(agg, x, scale, W1, b1.reshape(1, HID), W2, b2.reshape(1, EMB),
      g.reshape(1, EMB), be.reshape(1, EMB))


def _pool_body(x_ref, batch_ref, cmat_ref, o_ref, *, nrows, nblocks, B):
    i = pl.program_id(0)

    @pl.when(i == 0)
    def _():
        o_ref[...] = jnp.zeros_like(o_ref)

    rowid = i * B + lax.broadcasted_iota(jnp.int32, (B, 1), 0)
    xm = jnp.where(rowid < nrows, x_ref[...], 0.0)
    b = batch_ref[...]
    onehot = (b == lax.broadcasted_iota(jnp.int32, (1, NGRP), 1)).astype(jnp.float32)
    o_ref[...] += lax.dot_general(onehot, xm, (((0,), (0,)), ((), ())),
                                  preferred_element_type=jnp.float32)

    @pl.when(i == nblocks - 1)
    def _():
        o_ref[...] = o_ref[...] / jnp.maximum(cmat_ref[...], 1.0)


def _pool(x, batch2d, cmat, interpret=False):
    n = x.shape[0]
    B = 2048
    npad = pl.cdiv(n, B) * B
    nblocks = npad // B
    bpad = jnp.full((npad, 1), NGRP, jnp.int32).at[:n].set(batch2d)
    body = functools.partial(_pool_body, nrows=n, nblocks=nblocks, B=B)
    return pl.pallas_call(
        body,
        grid=(nblocks,),
        in_specs=[pl.BlockSpec((B, EMB), lambda i: (i, 0)),
                  pl.BlockSpec((B, 1), lambda i: (i, 0)),
                  pl.BlockSpec((NGRP, 128), lambda i: (0, 0))],
        out_specs=pl.BlockSpec((NGRP, EMB), lambda i: (0, 0)),
        out_shape=jax.ShapeDtypeStruct((NGRP, EMB), jnp.float32),
        interpret=interpret,
    )(x, bpad, cmat)


def kernel(ab_x, ab_edge_index, ab_batch, ba_x, ba_edge_index, ba_edge_attr,
           ba_batch, W1, b1, W2, b2, gamma, beta):
    n_ab = ab_x.shape[0]
    n_ba = ba_x.shape[0]
    ab_src, ab_dst = ab_edge_index[0], ab_edge_index[1]
    ba_src, ba_dst = ba_edge_index[0], ba_edge_index[1]
    ab_batch2 = ab_batch.reshape(-1, 1)
    ba_batch2 = ba_batch.reshape(-1, 1)

    _, cmat_ab = _counts(ab_batch2)
    _, cmat_ba = _counts(ba_batch2)
    scale_ab = _node_scale(ab_batch2, cmat_ab)
    scale_ba = _node_scale(ba_batch2, cmat_ba)

    s_attr = jax.ops.segment_sum(ba_edge_attr, ba_dst, num_segments=n_ba)

    node_h, edge_h = ab_x, ba_x
    for l in range(LAYERS):
        last_act = (l != LAYERS - 1)
        agg_ab = jax.ops.segment_sum(node_h[ab_src] + edge_h, ab_dst,
                                     num_segments=n_ab)
        agg_ba = jax.ops.segment_sum(edge_h[ba_src], ba_dst,
                                     num_segments=n_ba) + s_attr
        node_h = _mlp(agg_ab, node_h, scale_ab, W1[l], b1[l], W2[l], b2[l],
                      gamma[l], beta[l], last_act)
        edge_h = _mlp(agg_ba, edge_h, scale_ba, W1[l], b1[l], W2[l], b2[l],
                      gamma[l], beta[l], last_act)

    ab_repr = _pool(node_h, ab_batch2, cmat_ab)
    ba_repr = _pool(edge_h, ba_batch2, cmat_ba)
    return (ab_repr, ba_repr, node_h, edge_h)


# speedup vs baseline: 1.5764x; 1.4914x over previous
import functools

import jax
import jax.numpy as jnp
from jax import lax
from jax.experimental import pallas as pl
from jax.experimental.pallas import tpu as pltpu
from jax.experimental.pallas import tpu_sc as plsc

EMB = 128
HID = 256
NGRP = 256
LAYERS = 3

CS = 3200
G = 128


def _sc_mesh():
    return plsc.VectorSubcoreMesh(core_axis_name="c", subcore_axis_name="s")


def _permute_kernel(e, l_pad):
    assert e % 32 == 0
    per_w = e // 32
    nfull = per_w // G
    tail = per_w - nfull * G

    scr = [pltpu.VMEM((G,), jnp.int32)] * 4
    if tail:
        scr += [pltpu.VMEM((tail,), jnp.int32)] * 4

    @functools.partial(
        pl.kernel,
        out_type=(jax.ShapeDtypeStruct((l_pad,), jnp.int32),
                  jax.ShapeDtypeStruct((l_pad,), jnp.int32),
                  jax.ShapeDtypeStruct((l_pad,), jnp.int32)),
        mesh=_sc_mesh(),
        scratch_types=scr,
    )
    def k(src_h, rel_h, pos_h, srcg_h, relg_h, eidg_h, *bufs):
        cid = lax.axis_index("c")
        sid = lax.axis_index("s")
        w = cid * 16 + sid
        base_w = w * per_w

        def do_batch(base, n, bufp, bufa, bufb, bufe):
            pltpu.sync_copy(pos_h.at[pl.ds(base, n)], bufp)
            pltpu.sync_copy(src_h.at[pl.ds(base, n)], bufa)
            pltpu.sync_copy(rel_h.at[pl.ds(base, n)], bufb)
            for i in range(n // 16):
                bufe[pl.ds(i * 16, 16)] = (
                    base + i * 16 + lax.broadcasted_iota(jnp.int32, (16,), 0))
            pltpu.sync_copy(bufa, srcg_h.at[bufp])
            pltpu.sync_copy(bufb, relg_h.at[bufp])
            pltpu.sync_copy(bufe, eidg_h.at[bufp])

        def body(j, carry):
            do_batch(base_w + j * G, G, *bufs[:4])
            return carry

        lax.fori_loop(0, nfull, body, 0)
        if tail:
            do_batch(base_w + nfull * G, tail, *bufs[4:8])

    return k


def _segsum_ba_kernel(t_rows, l_pad, n_out, init_small):
    nc = n_out // CS
    assert n_out % CS == 0 and nc % 2 == 0
    rows_t = CS // 16

    @functools.partial(
        pl.kernel,
        out_type=jax.ShapeDtypeStruct((n_out, EMB), jnp.float32),
        mesh=_sc_mesh(),
        scratch_types=[
            pltpu.VMEM((128,), jnp.int32),
            pltpu.VMEM((G,), jnp.int32),
            pltpu.VMEM((G,), jnp.int32),
            pltpu.VMEM((G, EMB), jnp.float32),
            pltpu.VMEM_SHARED((CS + 1, EMB), jnp.float32),
            pltpu.SemaphoreType.DMA,
        ],
    )
    def k(table_h, idxg_h, relg_h, tab_h, init_h, out_h,
          tab_v, idx_v, rel_v, rows_v, acc, sem):
        cid = lax.axis_index("c")
        sid = lax.axis_index("s")
        pltpu.sync_copy(tab_h, tab_v)

        for cl in range(nc // 2):
            c = cl * 2 + cid
            g0 = (2 * cl) // 16
            l0 = (2 * cl) % 16
            sg = tab_v[pl.ds(g0 * 16, 16)]
            cg = tab_v[pl.ds(64 + g0 * 16, 16)]
            start = pl.multiple_of(jnp.where(cid == 1, sg[l0 + 1], sg[l0]), G)
            cnt = jnp.where(cid == 1, cg[l0 + 1], cg[l0])
            if init_small:
                pltpu.sync_copy(init_h,
                                acc.at[pl.ds(sid * rows_t, rows_t)])
            else:
                pltpu.sync_copy(
                    init_h.at[pl.ds(c * CS + sid * rows_t, rows_t)],
                    acc.at[pl.ds(sid * rows_t, rows_t)])
            plsc.subcore_barrier()

            nb = (cnt + (G - 1)) // G
            trip = (jnp.maximum(nb - sid, 0) + 15) // 16

            def body(m, carry):
                j = sid + 16 * m
                base = start + j * G
                pltpu.sync_copy(idxg_h.at[pl.ds(base, G)], idx_v)
                pltpu.sync_copy(relg_h.at[pl.ds(base, G)], rel_v)
                for i in range(G // 16):
                    off = (j * G + i * 16
                           + lax.broadcasted_iota(jnp.int32, (16,), 0))
                    valid = off < cnt
                    sl = pl.ds(i * 16, 16)
                    idx_v[sl] = jnp.where(valid, idx_v[sl], 0)
                    rel_v[sl] = jnp.where(valid, rel_v[sl], CS)
                pltpu.async_copy(table_h.at[idx_v], rows_v, sem).wait()
                pltpu.sync_copy(rows_v, acc.at[rel_v], add=True)
                return carry

            lax.fori_loop(0, trip, body, 0)
            plsc.subcore_barrier()
            pltpu.sync_copy(acc.at[pl.ds(sid * rows_t, rows_t)],
                            out_h.at[pl.ds(c * CS + sid * rows_t, rows_t)])
            plsc.subcore_barrier()

    return k


def _segsum_ab_kernel(e_ab, n_ab):
    assert e_ab % 32 == 0
    per_w = e_ab // 32
    nfull = per_w // G
    tail = per_w - nfull * G
    n_pad = ((n_ab + 127) // 128) * 128
    rows_t = n_pad // 16

    scr = [
        pltpu.VMEM((G,), jnp.int32),
        pltpu.VMEM((G,), jnp.int32),
        pltpu.VMEM((G, EMB), jnp.float32),
        pltpu.VMEM((G, EMB), jnp.float32),
        pltpu.VMEM_SHARED((n_pad, EMB), jnp.float32),
        pltpu.SemaphoreType.DMA,
    ]
    if tail:
        scr += [pltpu.VMEM((tail,), jnp.int32),
                pltpu.VMEM((tail,), jnp.int32)]

    @functools.partial(
        pl.kernel,
        out_type=(jax.ShapeDtypeStruct((n_pad, EMB), jnp.float32),
                  jax.ShapeDtypeStruct((n_pad, EMB), jnp.float32)),
        mesh=_sc_mesh(),
        scratch_types=scr,
    )
    def k(node_h, edge_h, src_h, dst_h, zsmall_h, out0_h, out1_h,
          idx_v, rel_v, rows_v, rows2_v, acc, sem, *tailbufs):
        cid = lax.axis_index("c")
        sid = lax.axis_index("s")
        base_w = (cid * 16 + sid) * per_w

        zrows = zsmall_h.shape[0]
        for i in range(rows_t // zrows):
            pltpu.sync_copy(
                zsmall_h, acc.at[pl.ds(sid * rows_t + i * zrows, zrows)])
        rem = rows_t - (rows_t // zrows) * zrows
        if rem:
            pltpu.sync_copy(zsmall_h.at[pl.ds(0, rem)],
                            acc.at[pl.ds(sid * rows_t
                                         + (rows_t // zrows) * zrows, rem)])
        plsc.subcore_barrier()

        def do_batch(base, n, bi, br):
            bro = rows_v.at[pl.ds(0, n)] if n != G else rows_v
            bro2 = rows2_v.at[pl.ds(0, n)] if n != G else rows2_v
            pltpu.sync_copy(src_h.at[pl.ds(base, n)], bi)
            pltpu.sync_copy(dst_h.at[pl.ds(base, n)], br)
            pltpu.async_copy(node_h.at[bi], bro, sem).wait()
            pltpu.sync_copy(edge_h.at[pl.ds(base, n)], bro2)
            pltpu.sync_copy(bro, acc.at[br], add=True)
            pltpu.sync_copy(bro2, acc.at[br], add=True)

        def body(j, carry):
            do_batch(base_w + j * G, G, idx_v, rel_v)
            return carry

        lax.fori_loop(0, nfull, body, 0)
        if tail:
            do_batch(base_w + nfull * G, tail, tailbufs[0], tailbufs[1])
        plsc.subcore_barrier()

        out = [out0_h, out1_h]
        for k2 in range(2):
            @pl.when(cid == k2)
            def _():
                pltpu.sync_copy(acc.at[pl.ds(sid * rows_t, rows_t)],
                                out[k2].at[pl.ds(sid * rows_t, rows_t)])
        plsc.subcore_barrier()

    return k


def _counts_body(batch_ref, row_ref, col_ref):
    i = pl.program_id(0)

    @pl.when(i == 0)
    def _():
        row_ref[...] = jnp.zeros_like(row_ref)
        col_ref[...] = jnp.zeros_like(col_ref)

    b = batch_ref[...]
    onehot = (b == lax.broadcasted_iota(jnp.int32, (1, NGRP), 1)).astype(jnp.float32)
    row_ref[0:1, :] += jnp.sum(onehot, axis=0, keepdims=True)
    col_ref[...] += lax.dot_general(
        onehot, jnp.ones((onehot.shape[0], 128), jnp.float32),
        (((0,), (0,)), ((), ())), preferred_element_type=jnp.float32)


def _counts(batch2d):
    n = batch2d.shape[0]
    B = 2048
    npad = pl.cdiv(n, B) * B
    bpad = jnp.full((npad, 1), NGRP, jnp.int32).at[:n].set(batch2d)
    return pl.pallas_call(
        _counts_body,
        grid=(npad // B,),
        in_specs=[pl.BlockSpec((B, 1), lambda i: (i, 0))],
        out_specs=[pl.BlockSpec((8, NGRP), lambda i: (0, 0)),
                   pl.BlockSpec((NGRP, 128), lambda i: (0, 0))],
        out_shape=[jax.ShapeDtypeStruct((8, NGRP), jnp.float32),
                   jax.ShapeDtypeStruct((NGRP, 128), jnp.float32)],
    )(bpad)


def _scale_body(batch_ref, cmat_ref, o_ref):
    b = batch_ref[...]
    onehot = (b == lax.broadcasted_iota(jnp.int32, (1, NGRP), 1)).astype(jnp.float32)
    rs = lax.rsqrt(jnp.maximum(cmat_ref[...][:, 0:1], 1.0))
    o_ref[...] = lax.dot_general(onehot, rs, (((1,), (0,)), ((), ())),
                                 preferred_element_type=jnp.float32)


def _node_scale(batch2d, cmat):
    n = batch2d.shape[0]
    B = 2048
    npad = pl.cdiv(n, B) * B
    bpad = jnp.full((npad, 1), NGRP, jnp.int32).at[:n].set(batch2d)
    out = pl.pallas_call(
        _scale_body,
        grid=(npad // B,),
        in_specs=[pl.BlockSpec((B, 1), lambda i: (i, 0)),
                  pl.BlockSpec((NGRP, 128), lambda i: (0, 0))],
        out_specs=pl.BlockSpec((B, 1), lambda i: (i, 0)),
        out_shape=jax.ShapeDtypeStruct((npad, 1), jnp.float32),
    )(bpad, cmat)
    return out[:n]


def _mlp_body(two_agg, agg_ref, agg2_ref, x_ref, scale_ref, W1_ref, b1_ref,
              W2_ref, b2_ref, g_ref, be_ref, o_ref, *, last_act):
    agg = agg_ref[...]
    if two_agg:
        agg = agg + agg2_ref[...]
    u = lax.dot_general(agg, W1_ref[...], (((1,), (0,)), ((), ())),
                        preferred_element_type=jnp.float32) + b1_ref[...]
    u = jnp.maximum(u, 0.0)
    h = lax.dot_general(u, W2_ref[...], (((1,), (0,)), ((), ())),
                        preferred_element_type=jnp.float32) + b2_ref[...]
    mu = jnp.mean(h, axis=1, keepdims=True)
    var = jnp.mean((h - mu) * (h - mu), axis=1, keepdims=True)
    h = (h - mu) * lax.rsqrt(var + 1e-5) * g_ref[...] + be_ref[...]
    h = h * scale_ref[...]
    if last_act:
        h = jnp.maximum(h, 0.0)
    o_ref[...] = h + x_ref[...]


def _mlp(aggs, x, scale, W1, b1, W2, b2, g, be, last_act):
    n = x.shape[0]
    B = 1024
    two = len(aggs) == 2
    body = functools.partial(_mlp_body, two, last_act=last_act)
    if not two:
        def body(agg_ref, *rest, _b=functools.partial(_mlp_body, False,
                                                      last_act=last_act)):
            _b(agg_ref, agg_ref, *rest)
    row = pl.BlockSpec((B, EMB), lambda i: (i, 0))
    return pl.pallas_call(
        body,
        grid=(pl.cdiv(n, B),),
        in_specs=([row] * (2 if two else 1)
                  + [row,
                     pl.BlockSpec((B, 1), lambda i: (i, 0)),
                     pl.BlockSpec((EMB, HID), lambda i: (0, 0)),
                     pl.BlockSpec((1, HID), lambda i: (0, 0)),
                     pl.BlockSpec((HID, EMB), lambda i: (0, 0)),
                     pl.BlockSpec((1, EMB), lambda i: (0, 0)),
                     pl.BlockSpec((1, EMB), lambda i: (0, 0)),
                     pl.BlockSpec((1, EMB), lambda i: (0, 0))]),
        out_specs=row,
        out_shape=jax.ShapeDtypeStruct((n, EMB), jnp.float32),
    )(*aggs, x, scale, W1, b1.reshape(1, HID), W2, b2.reshape(1, EMB),
      g.reshape(1, EMB), be.reshape(1, EMB))


def _pool_body(x_ref, batch_ref, cmat_ref, o_ref, *, nrows, nblocks, B):
    i = pl.program_id(0)

    @pl.when(i == 0)
    def _():
        o_ref[...] = jnp.zeros_like(o_ref)

    rowid = i * B + lax.broadcasted_iota(jnp.int32, (B, 1), 0)
    xm = jnp.where(rowid < nrows, x_ref[...], 0.0)
    b = batch_ref[...]
    onehot = (b == lax.broadcasted_iota(jnp.int32, (1, NGRP), 1)).astype(jnp.float32)
    o_ref[...] += lax.dot_general(onehot, xm, (((0,), (0,)), ((), ())),
                                  preferred_element_type=jnp.float32)

    @pl.when(i == nblocks - 1)
    def _():
        o_ref[...] = o_ref[...] / jnp.maximum(cmat_ref[...], 1.0)


def _pool(x, batch2d, cmat):
    n = x.shape[0]
    B = 2048
    npad = pl.cdiv(n, B) * B
    nblocks = npad // B
    bpad = jnp.full((npad, 1), NGRP, jnp.int32).at[:n].set(batch2d)
    body = functools.partial(_pool_body, nrows=n, nblocks=nblocks, B=B)
    return pl.pallas_call(
        body,
        grid=(nblocks,),
        in_specs=[pl.BlockSpec((B, EMB), lambda i: (i, 0)),
                  pl.BlockSpec((B, 1), lambda i: (i, 0)),
                  pl.BlockSpec((NGRP, 128), lambda i: (0, 0))],
        out_specs=pl.BlockSpec((NGRP, EMB), lambda i: (0, 0)),
        out_shape=jax.ShapeDtypeStruct((NGRP, EMB), jnp.float32),
    )(x, bpad, cmat)


def kernel(ab_x, ab_edge_index, ab_batch, ba_x, ba_edge_index, ba_edge_attr,
           ba_batch, W1, b1, W2, b2, gamma, beta):
    n_ab = ab_x.shape[0]
    n_ba = ba_x.shape[0]
    e_ab = ab_edge_index.shape[1]
    e_ba = ba_edge_index.shape[1]
    ab_src, ab_dst = ab_edge_index[0], ab_edge_index[1]
    ba_src, ba_dst = ba_edge_index[0], ba_edge_index[1]
    ab_batch2 = ab_batch.reshape(-1, 1)
    ba_batch2 = ba_batch.reshape(-1, 1)
    nc = n_ba // CS

    c_e = ba_dst // CS
    onehot = (c_e[:, None] == jnp.arange(nc, dtype=jnp.int32)[None, :])
    oh32 = onehot.astype(jnp.int32)
    rank = jnp.sum((jnp.cumsum(oh32, axis=0) - oh32) * oh32, axis=1)
    cnt = jnp.sum(oh32, axis=0)
    cnt_pad = ((cnt + (G - 1)) // G) * G
    starts = jnp.concatenate(
        [jnp.zeros((1,), jnp.int32), jnp.cumsum(cnt_pad)[:-1]]).astype(jnp.int32)
    pos = jnp.sum(starts[None, :] * oh32, axis=1) + rank
    rel = (ba_dst - c_e * CS).astype(jnp.int32)
    tab = jnp.zeros((128,), jnp.int32).at[0:nc].set(starts).at[64:64 + nc].set(cnt)
    l_pad = e_ba + nc * G

    src_g, rel_g, eid_g = _permute_kernel(e_ba, l_pad)(
        ba_src, rel, pos.astype(jnp.int32))

    zsmall = jnp.zeros((CS // 16, EMB), jnp.float32)
    s_attr = _segsum_ba_kernel(e_ba, l_pad, n_ba, True)(
        ba_edge_attr, eid_g, rel_g, tab, zsmall)

    _, cmat_ab = _counts(ab_batch2)
    _, cmat_ba = _counts(ba_batch2)
    scale_ab = _node_scale(ab_batch2, cmat_ab)
    scale_ba = _node_scale(ba_batch2, cmat_ba)

    ab_kern = _segsum_ab_kernel(e_ab, n_ab)
    ba_kern = _segsum_ba_kernel(n_ba, l_pad, n_ba, False)
    n_pad_ab = ((n_ab + 127) // 128) * 128
    zsmall_ab = jnp.zeros((n_pad_ab // 16, EMB), jnp.float32)

    node_h, edge_h = ab_x, ba_x
    for l in range(LAYERS):
        last_act = (l != LAYERS - 1)
        agg_ab0, agg_ab1 = ab_kern(node_h, edge_h, ab_src, ab_dst, zsmall_ab)
        agg_ba = ba_kern(edge_h, src_g, rel_g, tab, s_attr)
        node_h = _mlp((agg_ab0, agg_ab1), node_h, scale_ab, W1[l], b1[l],
                      W2[l], b2[l], gamma[l], beta[l], last_act)
        edge_h = _mlp((agg_ba,), edge_h, scale_ba, W1[l], b1[l], W2[l],
                      b2[l], gamma[l], beta[l], last_act)

    ab_repr = _pool(node_h, ab_batch2, cmat_ab)
    ba_repr = _pool(edge_h, ba_batch2, cmat_ba)
    return (ab_repr, ba_repr, node_h, edge_h)
